# baseline (speedup 1.0000x reference)
import jax
import jax.numpy as jnp
from jax import lax
from jax.experimental import pallas as pl
from jax.experimental.pallas import tpu as pltpu
from jax.experimental.pallas import tpu_sc as plsc

TOKEN_SEQ_LEN = 1024
W = 1536
N = 64 * 1024
RPW_PROBE = 2048
NC, NS, L = 2, 16, 16
NW = NC * NS
TR = TOKEN_SEQ_LEN // NW
ZROW = TR
SEG = 2048
NSEG = N // SEG
DEPTH = 64


def _sc_push(pos_hbm, mask_hbm, table_hbm, out_hbm,
             tblbuf, zbuf, pseg, mseg, olist, zlist, cntbuf, wsem):
    wid = lax.axis_index("s") * NC + lax.axis_index("c")
    r0 = wid * TR
    base_probe = (wid * RPW_PROBE,)

    pltpu.sync_copy(table_hbm.at[pl.ds(r0, TR)], tblbuf)

    def zero_body(r):
        def zcol(g):
            zbuf[r, pl.ds(g * L, L)] = jnp.zeros((L,), jnp.float32)

        pl.loop(0, W // L)(zcol)

    pl.loop(0, L)(zero_body)

    def fire(j):
        v = olist[pl.ds(j, L)][0]
        o = v & jnp.int32(0xFFFF)
        rl = lax.shift_right_logical(v, jnp.int32(16))
        pltpu.async_copy(tblbuf.at[pl.ds(rl, 1)],
                         out_hbm.at[pl.ds(o, 1)], wsem)

    def drain_one():
        pltpu.make_async_copy(table_hbm.at[pl.ds(0, 1)],
                              tblbuf.at[pl.ds(0, 1)], wsem).wait()

    def drain_16():
        pltpu.make_async_copy(table_hbm.at[pl.ds(0, L)],
                              zbuf, wsem).wait()

    def seg_body(s, inflight):
        pltpu.sync_copy(pos_hbm.at[pl.ds(s * SEG, SEG)], pseg)
        pltpu.sync_copy(mask_hbm.at[pl.ds(s * SEG, SEG)], mseg)

        lanes = lax.iota(jnp.int32, L)
        obase = s * SEG + lanes

        def scan_body(i, offv):
            sl = pl.ds(i * L, L)
            p = pseg[sl]
            m = mseg[sl]
            q = jnp.where(m != jnp.int32(0), p - r0, jnp.int32(-1))
            inr = plsc.bitcast(q, jnp.uint32) < jnp.uint32(TR)
            val = (obase + i * L) | lax.shift_left(p - r0, jnp.int32(16))
            key = jnp.where(inr, jnp.uint32(0), jnp.uint32(1))
            _, vs = plsc.sort_key_val(key, val)
            plsc.store_scatter(olist, [offv + lanes], vs)
            return offv + plsc.all_reduce_population_count(inr)

        offv = lax.fori_loop(0, SEG // L, scan_body,
                             jnp.zeros((L,), jnp.int32))
        cntbuf[pl.ds(0, L)] = offv
        cnt = cntbuf[pl.ds(0, L)][0]

        def fire_body(j, fly):
            fire(j)

            @pl.when(fly >= DEPTH)
            def _():
                drain_one()

            return jnp.minimum(fly + 1, DEPTH)

        fly = lax.fori_loop(0, cnt, fire_body, inflight)

        @pl.when(s == wid)
        def _():
            def zscan_body(i, offv):
                sl = pl.ds(i * L, L)
                m = mseg[sl]
                zm = m == jnp.int32(0)
                key = jnp.where(zm, jnp.uint32(0), jnp.uint32(1))
                _, vs = plsc.sort_key_val(key, obase + i * L)
                plsc.store_scatter(zlist, [offv + lanes], vs)
                return offv + plsc.all_reduce_population_count(zm)

            zoffv = lax.fori_loop(0, SEG // L, zscan_body,
                                  jnp.zeros((L,), jnp.int32))
            cntbuf[pl.ds(0, L)] = zoffv
            zcnt = cntbuf[pl.ds(0, L)][0]

            ngz = lax.shift_right_logical(zcnt, jnp.int32(4))
            remz = zcnt & jnp.int32(15)

            def zloop(g, z):
                zv = zlist[pl.ds(g * L, L)]
                pltpu.async_copy(zbuf, out_hbm.at[zv], wsem)

                @pl.when(g >= 1)
                def _():
                    drain_16()

                return z

            lax.fori_loop(0, ngz, zloop, jnp.int32(0))

            zvt = zlist[pl.ds(ngz * L, L)]
            for t in range(L):
                @pl.when(t < remz)
                def _(t=t):
                    pltpu.async_copy(zbuf.at[pl.ds(0, 1)],
                                     out_hbm.at[pl.ds(zvt[t], 1)], wsem)

            @pl.when(ngz >= 1)
            def _():
                drain_16()

            for t in range(L):
                @pl.when(t < remz)
                def _():
                    drain_one()

        return fly

    inflight = lax.fori_loop(0, NSEG, seg_body, jnp.int32(0))

    def tail_body(t):
        @pl.when(t < inflight)
        def _():
            drain_one()

    pl.loop(0, DEPTH)(tail_body)


@jax.jit
def kernel(obs_pos, obs_mask, table):
    B, S = obs_pos.shape
    pos = obs_pos.reshape(N)
    mask = obs_mask.reshape(N)

    mesh = plsc.VectorSubcoreMesh(
        core_axis_name="c", subcore_axis_name="s",
        num_cores=NC, num_subcores=NS)
    out = pl.kernel(
        _sc_push,
        out_type=jax.ShapeDtypeStruct((N, W), jnp.float32),
        mesh=mesh,
        compiler_params=pltpu.CompilerParams(needs_layout_passes=False),
        scratch_types=[
            pltpu.VMEM((TR, W), jnp.float32),
            pltpu.VMEM((L, W), jnp.float32),
            pltpu.VMEM((SEG,), jnp.int32),
            pltpu.VMEM((SEG,), jnp.int32),
            pltpu.VMEM((SEG + L,), jnp.int32),
            pltpu.VMEM((SEG + L,), jnp.int32),
            pltpu.VMEM((L,), jnp.int32),
            pltpu.SemaphoreType.DMA,
        ],
    )(pos, mask, table)
    return out.reshape(B, S, W)


# speedup vs baseline: 1.0019x; 1.0019x over previous
import jax
import jax.numpy as jnp
from jax import lax
from jax.experimental import pallas as pl
from jax.experimental.pallas import tpu as pltpu
from jax.experimental.pallas import tpu_sc as plsc

TOKEN_SEQ_LEN = 1024
W = 1536
N = 64 * 1024
NC, NS, L = 2, 16, 16
NW = NC * NS
TR = TOKEN_SEQ_LEN // NW
SEG = 2048
NSEG = N // SEG
DEPTH = 64


def _sc_push(pos_hbm, mask_hbm, table_hbm, out_hbm,
             tblbuf, zbuf, pseg, mseg, olist, zlist, cntbuf, wsem):
    wid = lax.axis_index("s") * NC + lax.axis_index("c")
    r0 = wid * TR

    pltpu.sync_copy(table_hbm.at[pl.ds(r0, TR)], tblbuf)

    def zero_body(r):
        def zcol(g):
            zbuf[r, pl.ds(g * L, L)] = jnp.zeros((L,), jnp.float32)

        pl.loop(0, W // L)(zcol)

    pl.loop(0, L)(zero_body)

    def fire(j):
        v = olist[pl.ds(j, L)][0]
        o = v & jnp.int32(0xFFFF)
        rl = lax.shift_right_logical(v, jnp.int32(16))
        pltpu.async_copy(tblbuf.at[pl.ds(rl, 1)],
                         out_hbm.at[pl.ds(o, 1)], wsem)

    def drain_one():
        pltpu.make_async_copy(table_hbm.at[pl.ds(0, 1)],
                              tblbuf.at[pl.ds(0, 1)], wsem).wait()

    def drain_16():
        pltpu.make_async_copy(table_hbm.at[pl.ds(0, L)],
                              zbuf, wsem).wait()

    def seg_body(s, inflight):
        pltpu.sync_copy(pos_hbm.at[pl.ds(s * SEG, SEG)], pseg)
        pltpu.sync_copy(mask_hbm.at[pl.ds(s * SEG, SEG)], mseg)

        lanes = lax.iota(jnp.int32, L)
        obase = s * SEG + lanes

        def scan_body(i, offv):
            sl = pl.ds(i * L, L)
            p = pseg[sl]
            m = mseg[sl]
            q = jnp.where(m != jnp.int32(0), p - r0, jnp.int32(-1))
            inr = plsc.bitcast(q, jnp.uint32) < jnp.uint32(TR)
            val = (obase + i * L) | lax.shift_left(p - r0, jnp.int32(16))
            key = jnp.where(inr, jnp.uint32(0), jnp.uint32(1))
            _, vs = plsc.sort_key_val(key, val)
            plsc.store_scatter(olist, [offv + lanes], vs)
            return offv + plsc.all_reduce_population_count(inr)

        offv = lax.fori_loop(0, SEG // L, scan_body,
                             jnp.zeros((L,), jnp.int32))
        cntbuf[pl.ds(0, L)] = offv
        cnt = cntbuf[pl.ds(0, L)][0]

        def fire_body(j, fly):
            fire(j)

            @pl.when(fly >= DEPTH)
            def _():
                drain_one()

            return jnp.minimum(fly + 1, DEPTH)

        fly = lax.fori_loop(0, cnt, fire_body, inflight)

        @pl.when(s == wid)
        def _():
            def zscan_body(i, offv):
                sl = pl.ds(i * L, L)
                m = mseg[sl]
                zm = m == jnp.int32(0)
                key = jnp.where(zm, jnp.uint32(0), jnp.uint32(1))
                _, vs = plsc.sort_key_val(key, obase + i * L)
                plsc.store_scatter(zlist, [offv + lanes], vs)
                return offv + plsc.all_reduce_population_count(zm)

            zoffv = lax.fori_loop(0, SEG // L, zscan_body,
                                  jnp.zeros((L,), jnp.int32))
            cntbuf[pl.ds(0, L)] = zoffv
            zcnt = cntbuf[pl.ds(0, L)][0]

            ngz = lax.shift_right_logical(zcnt, jnp.int32(4))
            remz = zcnt & jnp.int32(15)

            def zloop(g, z):
                zv = zlist[pl.ds(g * L, L)]
                pltpu.async_copy(zbuf, out_hbm.at[zv], wsem)

                @pl.when(g >= 1)
                def _():
                    drain_16()

                return z

            lax.fori_loop(0, ngz, zloop, jnp.int32(0))

            zvt = zlist[pl.ds(ngz * L, L)]
            for t in range(L):
                @pl.when(t < remz)
                def _(t=t):
                    pltpu.async_copy(zbuf.at[pl.ds(0, 1)],
                                     out_hbm.at[pl.ds(zvt[t], 1)], wsem)

            @pl.when(ngz >= 1)
            def _():
                drain_16()

            for t in range(L):
                @pl.when(t < remz)
                def _():
                    drain_one()

        return fly

    inflight = lax.fori_loop(0, NSEG, seg_body, jnp.int32(0))

    def tail_body(t):
        @pl.when(t < inflight)
        def _():
            drain_one()

    pl.loop(0, DEPTH)(tail_body)


@jax.jit
def kernel(obs_pos, obs_mask, table):
    B, S = obs_pos.shape
    pos = obs_pos.reshape(N)
    mask = obs_mask.reshape(N)

    mesh = plsc.VectorSubcoreMesh(
        core_axis_name="c", subcore_axis_name="s",
        num_cores=NC, num_subcores=NS)
    out = pl.kernel(
        _sc_push,
        out_type=jax.ShapeDtypeStruct((N, W), jnp.float32),
        mesh=mesh,
        compiler_params=pltpu.CompilerParams(needs_layout_passes=False),
        scratch_types=[
            pltpu.VMEM((TR, W), jnp.float32),
            pltpu.VMEM((L, W), jnp.float32),
            pltpu.VMEM((SEG,), jnp.int32),
            pltpu.VMEM((SEG,), jnp.int32),
            pltpu.VMEM((SEG + L,), jnp.int32),
            pltpu.VMEM((SEG + L,), jnp.int32),
            pltpu.VMEM((L,), jnp.int32),
            pltpu.SemaphoreType.DMA,
        ],
    )(pos, mask, table)
    return out.reshape(B, S, W)
